# Initial kernel scaffold; baseline (speedup 1.0000x reference)
import jax
import jax.numpy as jnp
from jax.experimental import pallas as pl


def kernel(x, W1, W2, b2, W3, b3, Wc, bc, Wo1, bo1, Wo2, bo2):
    raise NotImplementedError("write your pallas kernel here")


# speedup vs baseline: 71.5290x; 71.5290x over previous
import functools

import jax
import jax.numpy as jnp
import numpy as np
from jax.experimental import pallas as pl
from jax.experimental.pallas import tpu as pltpu

INPUT_FEAT = 32
NHIDDEN = 16
K = 8
PARTS = 128
QUAD = 4


def _elu(x):
    return jnp.where(x > 0, x, jnp.exp(jnp.minimum(x, 0.0)) - 1.0)


def _body(x_ref, w1q_ref, w2q_ref, b2q_ref, w3q_ref, b3q_ref,
          uq_ref, vq_ref, bc_ref, pool_ref, wo1t_ref, bo1_ref,
          wo2t_ref, bo2_ref, out_ref, *, G):
    f32 = jnp.float32
    dot = lambda a, b: jax.lax.dot_general(
        a, b, (((a.ndim - 1,), (0,)), ((), ())), preferred_element_type=f32)

    xb = x_ref[:].reshape(G * PARTS // QUAD, QUAD * INPUT_FEAT)
    h = _elu(dot(xb, w1q_ref[:]))
    h = _elu(dot(h, w2q_ref[:]) + b2q_ref[:])
    h = _elu(dot(h, w3q_ref[:]) + b3q_ref[:])
    u2 = dot(h, uq_ref[:])
    v2 = dot(h, vq_ref[:])

    def degrade(a):
        a3 = a.reshape(G, PARTS // QUAD, QUAD * NHIDDEN)
        return jnp.concatenate(
            [a3[:, :, q * NHIDDEN:(q + 1) * NHIDDEN] for q in range(QUAD)],
            axis=1)

    hb = degrade(h)
    vb = degrade(v2)

    gram = jax.lax.dot_general(hb, hb, (((2,), (2,)), ((0,), (0,))),
                               preferred_element_type=f32)
    eye = (jax.lax.broadcasted_iota(jnp.int32, (PARTS, PARTS), 0) ==
           jax.lax.broadcasted_iota(jnp.int32, (PARTS, PARTS), 1)).astype(f32)
    ge = gram * eye
    d_col = jnp.sum(ge, axis=2, keepdims=True)
    d_row = jnp.sum(ge, axis=1, keepdims=True)
    dist = d_col + d_row - 2.0 * gram

    cur = dist
    maxv = jnp.full((G, PARTS, NHIDDEN), -jnp.inf, dtype=f32)
    big = jnp.float32(jnp.finfo(jnp.float32).max)
    for _ in range(K):
        rowmin = jnp.min(cur, axis=2, keepdims=True)
        hit = cur <= rowmin
        vt = jax.lax.dot_general(hit.astype(f32), vb,
                                 (((2,), (1,)), ((0,), (0,))),
                                 preferred_element_type=f32)
        maxv = jnp.maximum(maxv, vt)
        cur = jnp.where(hit, big, cur)

    ub = degrade(u2)
    agg = _elu(ub + maxv + bc_ref[:].reshape(1, 1, NHIDDEN))
    agg2 = agg.reshape(G * PARTS, NHIDDEN)
    pooled = dot(pool_ref[:], agg2)

    o = _elu(dot(pooled, wo1t_ref[:]) + bo1_ref[:])
    o = dot(o, wo2t_ref[:]) + bo2_ref[:]
    out_ref[:] = 1.0 / (1.0 + jnp.exp(-o))


@functools.partial(jax.jit, static_argnames=("G", "interpret"))
def _run(x, W1q, W2q, b2q, W3q, b3q, Uq, Vq, bc, pool, Wo1t, bo1, Wo2t, bo2,
         G=16, interpret=False):
    B = x.shape[0]
    grid = B // G
    full = lambda a: pl.BlockSpec(a.shape, lambda i: (0,) * a.ndim)
    return pl.pallas_call(
        functools.partial(_body, G=G),
        grid=(grid,),
        in_specs=[
            pl.BlockSpec((G, x.shape[1]), lambda i: (i, 0)),
            full(W1q), full(W2q), full(b2q), full(W3q), full(b3q),
            full(Uq), full(Vq), full(bc), full(pool),
            full(Wo1t), full(bo1), full(Wo2t), full(bo2),
        ],
        out_specs=pl.BlockSpec((G, 1), lambda i: (i, 0)),
        out_shape=jax.ShapeDtypeStruct((B, 1), jnp.float32),
        compiler_params=pltpu.CompilerParams(
            dimension_semantics=("parallel",),
        ),
        interpret=interpret,
    )(x, W1q, W2q, b2q, W3q, b3q, Uq, Vq, bc, pool, Wo1t, bo1, Wo2t, bo2)


def _quad_block_diag(wt):
    return jnp.kron(jnp.eye(QUAD, dtype=jnp.float32), wt)


def kernel(x, W1, W2, b2, W3, b3, Wc, bc, Wo1, bo1, Wo2, bo2, G=16):
    Wc1 = Wc[:, :NHIDDEN]
    Wc2 = Wc[:, NHIDDEN:]
    W1q = _quad_block_diag(jnp.transpose(W1))
    W2q = _quad_block_diag(jnp.transpose(W2))
    W3q = _quad_block_diag(jnp.transpose(W3))
    Uq = _quad_block_diag(jnp.transpose(Wc1 - Wc2))
    Vq = _quad_block_diag(jnp.transpose(Wc2))
    b2q = jnp.tile(b2.reshape(1, NHIDDEN), (1, QUAD))
    b3q = jnp.tile(b3.reshape(1, NHIDDEN), (1, QUAD))
    pool = jnp.asarray(
        np.kron(np.eye(G, dtype=np.float32),
                np.ones((1, PARTS), dtype=np.float32) / PARTS))
    return _run(x, W1q, W2q, b2q, W3q, b3q, Uq, Vq,
                bc.reshape(1, NHIDDEN), pool,
                jnp.transpose(Wo1), bo1.reshape(1, NHIDDEN // 2),
                jnp.transpose(Wo2), bo2.reshape(1, 1), G=G)


# speedup vs baseline: 106.6025x; 1.4903x over previous
import functools

import jax
import jax.numpy as jnp
from jax.experimental import pallas as pl
from jax.experimental.pallas import tpu as pltpu

INPUT_FEAT = 32
NHIDDEN = 16
K = 8
PARTS = 128


def _elu(x):
    return jnp.where(x > 0, x, jnp.exp(jnp.minimum(x, 0.0)) - 1.0)


def _body(x_ref, w1_ref, w2_ref, b2_ref, w3_ref, b3_ref,
          u_ref, v_ref, bc_ref, wo1_ref, bo1_ref, wo2_ref, bo2_ref,
          out_ref, *, G):
    f32 = jnp.float32
    bdot = lambda a, b: jax.lax.dot_general(
        a, b, (((2,), (1,)), ((0,), (0,))), preferred_element_type=f32)

    xb = x_ref[:]
    h = _elu(bdot(w1_ref[:], xb))
    h = _elu(bdot(w2_ref[:], h) + b2_ref[:])
    h = _elu(bdot(w3_ref[:], h) + b3_ref[:])
    uT = bdot(u_ref[:], h)
    vT = bdot(v_ref[:], h)

    gram = jax.lax.dot_general(h, h, (((1,), (1,)), ((0,), (0,))),
                               preferred_element_type=f32)
    eye = (jax.lax.broadcasted_iota(jnp.int32, (PARTS, PARTS), 0) ==
           jax.lax.broadcasted_iota(jnp.int32, (PARTS, PARTS), 1)).astype(f32)
    ge = gram * eye
    d_col = jnp.sum(ge, axis=2, keepdims=True)
    d_row = jnp.sum(ge, axis=1, keepdims=True)
    rdist = d_col - 2.0 * gram

    big = jnp.float32(jnp.finfo(jnp.float32).max)
    m = -d_row
    maxvT = vT
    for _ in range(K - 1):
        cand = jnp.where(rdist > m, rdist, big)
        m = jnp.min(cand, axis=1, keepdims=True)
        hitf = jnp.where(cand <= m, 1.0, 0.0)
        vtT = bdot(vT, hitf)
        maxvT = jnp.maximum(maxvT, vtT)

    aggT = _elu(uT + maxvT + bc_ref[:])
    z = bdot(wo1_ref[:], aggT)
    pooled = jnp.mean(z, axis=2, keepdims=True)
    o = _elu(pooled + bo1_ref[:])
    o = bdot(wo2_ref[:], o) + bo2_ref[:]
    out_ref[:] = (1.0 / (1.0 + jnp.exp(-o))).reshape(G, 1)


@functools.partial(jax.jit, static_argnames=("G", "interpret"))
def _run(x3, W1g, W2g, b2r, W3g, b3r, Ug, Vg, bcr, Wo1g, bo1r, Wo2g, bo2r,
         G=32, interpret=False):
    B = x3.shape[0]
    grid = B // G
    full = lambda a: pl.BlockSpec(a.shape, lambda i: (0,) * a.ndim)
    return pl.pallas_call(
        functools.partial(_body, G=G),
        grid=(grid,),
        in_specs=[
            pl.BlockSpec((G, INPUT_FEAT, PARTS), lambda i: (i, 0, 0)),
            full(W1g), full(W2g), full(b2r), full(W3g), full(b3r),
            full(Ug), full(Vg), full(bcr), full(Wo1g), full(bo1r),
            full(Wo2g), full(bo2r),
        ],
        out_specs=pl.BlockSpec((G, 1), lambda i: (i, 0)),
        out_shape=jax.ShapeDtypeStruct((B, 1), jnp.float32),
        compiler_params=pltpu.CompilerParams(
            dimension_semantics=("parallel",),
        ),
        interpret=interpret,
    )(x3, W1g, W2g, b2r, W3g, b3r, Ug, Vg, bcr, Wo1g, bo1r, Wo2g, bo2r)


def kernel(x, W1, W2, b2, W3, b3, Wc, bc, Wo1, bo1, Wo2, bo2, G=32):
    B = x.shape[0]
    x3 = jnp.transpose(x.reshape(B, PARTS, INPUT_FEAT), (0, 2, 1))
    tile = lambda w: jnp.broadcast_to(w[None], (G,) + w.shape)
    Wc1 = Wc[:, :NHIDDEN]
    Wc2 = Wc[:, NHIDDEN:]
    return _run(x3, tile(W1), tile(W2), b2.reshape(1, NHIDDEN, 1),
                tile(W3), b3.reshape(1, NHIDDEN, 1),
                tile(Wc1 - Wc2), tile(Wc2), bc.reshape(1, NHIDDEN, 1),
                tile(Wo1), bo1.reshape(1, NHIDDEN // 2, 1),
                tile(Wo2), bo2.reshape(1, 1, 1), G=G)


# speedup vs baseline: 109.3026x; 1.0253x over previous
import functools

import jax
import jax.numpy as jnp
from jax.experimental import pallas as pl
from jax.experimental.pallas import tpu as pltpu

INPUT_FEAT = 32
NHIDDEN = 16
K = 8
PARTS = 128
QUAD = 4


def _elu(x):
    return jnp.where(x > 0, x, jnp.exp(x) - 1.0)


def _body(x_ref, w1_ref, w2_ref, b2_ref, w3_ref, b3_ref,
          u_ref, v_ref, bc_ref, wo1_ref, bo1_ref, wo2_ref, bo2_ref,
          out_ref, *, G):
    f32 = jnp.float32
    bdot = lambda a, b: jax.lax.dot_general(
        a, b, (((2,), (1,)), ((0,), (0,))), preferred_element_type=f32)

    xq = x_ref[:]
    xb = jnp.concatenate(
        [jnp.swapaxes(xq[:, :, 32 * q:32 * (q + 1)], 1, 2)
         for q in range(QUAD)], axis=2)
    h = _elu(bdot(w1_ref[:], xb))
    h = _elu(bdot(w2_ref[:], h) + b2_ref[:])
    h = _elu(bdot(w3_ref[:], h) + b3_ref[:])
    uT = bdot(u_ref[:], h)
    vT = bdot(v_ref[:], h)

    gram = jax.lax.dot_general(h, h, (((1,), (1,)), ((0,), (0,))),
                               preferred_element_type=f32)
    eye = (jax.lax.broadcasted_iota(jnp.int32, (PARTS, PARTS), 0) ==
           jax.lax.broadcasted_iota(jnp.int32, (PARTS, PARTS), 1)).astype(f32)
    ge = gram * eye
    d_col = jnp.sum(ge, axis=2, keepdims=True)
    d_row = jnp.sum(ge, axis=1, keepdims=True)
    rdist = d_col - 2.0 * gram
    m0 = -d_row
    big = jnp.float32(jnp.finfo(jnp.float32).max)

    m = m0
    maxvT = vT
    for _ in range(K - 1):
        cand = jnp.where(rdist > m, rdist, big)
        m = jnp.min(cand, axis=1, keepdims=True)
        hitf = jnp.where(cand <= m, 1.0, 0.0)
        vtT = bdot(vT, hitf)
        maxvT = jnp.maximum(maxvT, vtT)

    aggT = _elu(uT + maxvT + bc_ref[:])
    z = bdot(wo1_ref[:], aggT)
    pooled = jnp.mean(z, axis=2, keepdims=True)
    o = _elu(pooled + bo1_ref[:])
    o = bdot(wo2_ref[:], o) + bo2_ref[:]
    out_ref[:] = (1.0 / (1.0 + jnp.exp(-o))).reshape(G, 1)


@functools.partial(jax.jit, static_argnames=("G", "interpret"))
def _run(x3, W1g, W2g, b2r, W3g, b3r, Ug, Vg, bcr, Wo1g, bo1r, Wo2g, bo2r,
         G=32, interpret=False):
    B = x3.shape[0]
    grid = B // G
    full = lambda a: pl.BlockSpec(a.shape, lambda i: (0,) * a.ndim)
    return pl.pallas_call(
        functools.partial(_body, G=G),
        grid=(grid,),
        in_specs=[
            pl.BlockSpec((G, PARTS // QUAD, QUAD * INPUT_FEAT),
                         lambda i: (i, 0, 0)),
            full(W1g), full(W2g), full(b2r), full(W3g), full(b3r),
            full(Ug), full(Vg), full(bcr), full(Wo1g), full(bo1r),
            full(Wo2g), full(bo2r),
        ],
        out_specs=pl.BlockSpec((G, 1), lambda i: (i, 0)),
        out_shape=jax.ShapeDtypeStruct((B, 1), jnp.float32),
        compiler_params=pltpu.CompilerParams(
            dimension_semantics=("parallel",),
        ),
        interpret=interpret,
    )(x3, W1g, W2g, b2r, W3g, b3r, Ug, Vg, bcr, Wo1g, bo1r, Wo2g, bo2r)


def kernel(x, W1, W2, b2, W3, b3, Wc, bc, Wo1, bo1, Wo2, bo2, G=64):
    tile = lambda w: jnp.broadcast_to(w[None], (G,) + w.shape)
    Wc1 = Wc[:, :NHIDDEN]
    Wc2 = Wc[:, NHIDDEN:]
    x = x.reshape(x.shape[0], PARTS // QUAD, QUAD * INPUT_FEAT)
    return _run(x, tile(W1), tile(W2), b2.reshape(1, NHIDDEN, 1),
                tile(W3), b3.reshape(1, NHIDDEN, 1),
                tile(Wc1 - Wc2), tile(Wc2), bc.reshape(1, NHIDDEN, 1),
                tile(Wo1), bo1.reshape(1, NHIDDEN // 2, 1),
                tile(Wo2), bo2.reshape(1, 1, 1), G=G)


# speedup vs baseline: 131.9878x; 1.2075x over previous
import functools

import jax
import jax.numpy as jnp
from jax.experimental import pallas as pl
from jax.experimental.pallas import tpu as pltpu

INPUT_FEAT = 32
NHIDDEN = 16
K = 8
PARTS = 128
QUAD = 4


def _elu(x):
    return jnp.where(x > 0, x, jnp.exp(x) - 1.0)


def _body(x_ref, w1_ref, w2_ref, b2_ref, w3_ref, b3_ref,
          u_ref, v_ref, bc_ref, wo1_ref, bo1_ref, wo2_ref, bo2_ref,
          out_ref, *, G):
    f32 = jnp.float32
    bdot = lambda a, b: jax.lax.dot_general(
        a, b, (((2,), (1,)), ((0,), (0,))), preferred_element_type=f32)
    wdot = lambda w, b: bdot(
        jnp.broadcast_to(w[:][None], (G,) + w.shape), b)

    xq = x_ref[:].reshape(G, PARTS // QUAD, QUAD * INPUT_FEAT)
    xb = jnp.concatenate(
        [jnp.swapaxes(xq[:, :, 32 * q:32 * (q + 1)], 1, 2)
         for q in range(QUAD)], axis=2)
    h = _elu(wdot(w1_ref, xb))
    h = _elu(wdot(w2_ref, h) + b2_ref[:])
    h = _elu(wdot(w3_ref, h) + b3_ref[:])
    uT = wdot(u_ref, h)
    vT = wdot(v_ref, h)

    gram = jax.lax.dot_general(h, h, (((1,), (1,)), ((0,), (0,))),
                               preferred_element_type=f32)
    eye = (jax.lax.broadcasted_iota(jnp.int32, (PARTS, PARTS), 0) ==
           jax.lax.broadcasted_iota(jnp.int32, (PARTS, PARTS), 1)).astype(f32)
    ge = gram * eye
    d_col = jnp.sum(ge, axis=2, keepdims=True)
    d_row = jnp.sum(ge, axis=1, keepdims=True)
    rdist = d_col - 2.0 * gram
    m0 = -d_row
    big = jnp.float32(jnp.finfo(jnp.float32).max)

    m = m0
    maxvT = vT
    for _ in range(K - 1):
        cand = jnp.where(rdist > m, rdist, big)
        m = jnp.min(cand, axis=1, keepdims=True)
        hitf = jnp.where(cand <= m, 1.0, 0.0)
        vtT = bdot(vT, hitf)
        maxvT = jnp.maximum(maxvT, vtT)

    aggT = _elu(uT + maxvT + bc_ref[:])
    z = wdot(wo1_ref, aggT)
    pooled = jnp.mean(z, axis=2, keepdims=True)
    o = _elu(pooled + bo1_ref[:])
    o = wdot(wo2_ref, o) + bo2_ref[:]
    out_ref[:] = (1.0 / (1.0 + jnp.exp(-o))).reshape(G, 1)


@functools.partial(jax.jit, static_argnames=("G", "interpret"))
def _run(x3, W1g, W2g, b2r, W3g, b3r, Ug, Vg, bcr, Wo1g, bo1r, Wo2g, bo2r,
         G=32, interpret=False):
    B = x3.shape[0]
    grid = B // G
    full = lambda a: pl.BlockSpec(a.shape, lambda i: (0,) * a.ndim)
    return pl.pallas_call(
        functools.partial(_body, G=G),
        grid=(grid,),
        in_specs=[
            pl.BlockSpec((G, PARTS * INPUT_FEAT), lambda i: (i, 0)),
            full(W1g), full(W2g), full(b2r), full(W3g), full(b3r),
            full(Ug), full(Vg), full(bcr), full(Wo1g), full(bo1r),
            full(Wo2g), full(bo2r),
        ],
        out_specs=pl.BlockSpec((G, 1), lambda i: (i, 0)),
        out_shape=jax.ShapeDtypeStruct((B, 1), jnp.float32),
        compiler_params=pltpu.CompilerParams(
            dimension_semantics=("parallel",),
        ),
        interpret=interpret,
    )(x3, W1g, W2g, b2r, W3g, b3r, Ug, Vg, bcr, Wo1g, bo1r, Wo2g, bo2r)


def kernel(x, W1, W2, b2, W3, b3, Wc, bc, Wo1, bo1, Wo2, bo2, G=64):
    Wc1 = Wc[:, :NHIDDEN]
    Wc2 = Wc[:, NHIDDEN:]
    return _run(x, W1, W2, b2.reshape(1, NHIDDEN, 1),
                W3, b3.reshape(1, NHIDDEN, 1),
                Wc1 - Wc2, Wc2, bc.reshape(1, NHIDDEN, 1),
                Wo1, bo1.reshape(1, NHIDDEN // 2, 1),
                Wo2, bo2.reshape(1, 1, 1), G=G)
